# baseline (speedup 1.0000x reference)
import functools

import jax
import jax.numpy as jnp
from jax import lax
from jax.experimental import pallas as pl
from jax.experimental.pallas import tpu as pltpu
from jax.experimental.pallas import tpu_sc as plsc

_BATCH = 16384
_DIM = 64
_NREL = 1000
_L = 16
_NC = 2
_NS = 16
_NW = _NC * _NS
_BPW = _BATCH // _NW
_G = 16
_NGRP = _BPW // _G


def _sc_body(h_hbm, r_hbm, t_hbm, ent_hbm, rel_hbm, out_hbm,
             hv, rv, tv, hbuf, tbuf, relv, ov, sem_a, sem_b):
    wid = lax.axis_index("s") * _NC + lax.axis_index("c")
    base = wid * _BPW

    pltpu.sync_copy(h_hbm.at[pl.ds(base, _BPW)], hv)
    pltpu.sync_copy(t_hbm.at[pl.ds(base, _BPW)], tv)
    pltpu.sync_copy(r_hbm.at[pl.ds(base, _BPW)], rv)
    pltpu.sync_copy(rel_hbm, relv)

    def fire(g, slot, sem):
        gsl = pl.ds(g * _G, _G)
        hg = hv[gsl] >> 3
        tg = tv[gsl] >> 3
        dst = pl.ds(slot * _G, _G)
        pltpu.async_copy(ent_hbm.at[hg], hbuf.at[dst], sem)
        pltpu.async_copy(ent_hbm.at[tg], tbuf.at[dst], sem)

    def drain(slot, sem):
        dst = pl.ds(slot * _G, _G)
        pltpu.make_async_copy(ent_hbm.at[pl.ds(0, _G)], hbuf.at[dst],
                              sem).wait()
        pltpu.make_async_copy(ent_hbm.at[pl.ds(0, _G)], tbuf.at[dst],
                              sem).wait()

    def compute(g, slot):
        gsl = pl.ds(g * _G, _G)
        hvv = hv[gsl]
        rvv = rv[gsl]
        tvv = tv[gsl]
        for n in range(_G):
            hs = (hvv[n] & 7) << 6
            ts = (tvv[n] & 7) << 6
            rw = rvv[n]
            rg = rw >> 3
            rs = (rw & 7) << 6
            row = slot * _G + n
            for c in range(_DIM // _L):
                sl = c * _L
                he = hbuf[row, pl.ds(pl.multiple_of(hs + sl, _L), _L)]
                te = tbuf[row, pl.ds(pl.multiple_of(ts + sl, _L), _L)]
                re = relv[rg, pl.ds(pl.multiple_of(rs + sl, _L), _L)]
                ov[n, pl.ds(sl, _L)] = he + re - te
        pltpu.sync_copy(ov, out_hbm.at[pl.ds(base + g * _G, _G)])

    fire(0, 0, sem_a)

    def body(m, carry):
        g0 = 2 * m
        fire(g0 + 1, 1, sem_b)
        drain(0, sem_a)
        compute(g0, 0)

        @pl.when(m < _NGRP // 2 - 1)
        def _():
            fire(g0 + 2, 0, sem_a)

        drain(1, sem_b)
        compute(g0 + 1, 1)
        return carry

    lax.fori_loop(0, _NGRP // 2, body, 0)


_trans_e = functools.partial(
    pl.kernel,
    mesh=plsc.VectorSubcoreMesh(core_axis_name="c", subcore_axis_name="s"),
    out_type=jax.ShapeDtypeStruct((_BATCH, _DIM), jnp.float32),
    scratch_types=[
        pltpu.VMEM((_BPW,), jnp.int32),
        pltpu.VMEM((_BPW,), jnp.int32),
        pltpu.VMEM((_BPW,), jnp.int32),
        pltpu.VMEM((2 * _G, 8 * _DIM), jnp.float32),
        pltpu.VMEM((2 * _G, 8 * _DIM), jnp.float32),
        pltpu.VMEM((_NREL // 8, 8 * _DIM), jnp.float32),
        pltpu.VMEM((_G, _DIM), jnp.float32),
        pltpu.SemaphoreType.DMA,
        pltpu.SemaphoreType.DMA,
    ],
)(_sc_body)


@jax.jit
def kernel(h, r, t, entity_embeddings, relation_embeddings):
    ent2 = entity_embeddings.reshape(-1, 8 * _DIM)
    rel2 = relation_embeddings.reshape(-1, 8 * _DIM)
    return _trans_e(
        h.astype(jnp.int32),
        r.astype(jnp.int32),
        t.astype(jnp.int32),
        ent2,
        rel2,
    )
